# baseline (speedup 1.0000x reference)
import functools

import jax
import jax.numpy as jnp
from jax import lax
from jax.experimental import pallas as pl
from jax.experimental.pallas import tpu as pltpu
from jax.experimental.pallas import tpu_sc as plsc

BATCH = 16384
EMB = 64
CAT_EMB = 32

_NC = 2
_NS = 16
_NW = _NC * _NS
_B_PER_W = BATCH // _NW
_NSEM = 8

_sc_mesh = plsc.VectorSubcoreMesh(core_axis_name="c", subcore_axis_name="s")


@functools.partial(
    pl.kernel,
    out_type=[
        jax.ShapeDtypeStruct((BATCH, EMB), jnp.float32),
        jax.ShapeDtypeStruct((BATCH, CAT_EMB), jnp.float32),
    ],
    mesh=_sc_mesh,
    scratch_types=(
        [pltpu.VMEM((_B_PER_W,), jnp.int32),
         pltpu.VMEM((_B_PER_W,), jnp.int32)]
        + [pltpu.SemaphoreType.DMA] * _NSEM
    ),
)
def _sc_gather(ids_hbm, cids_hbm, item_table_hbm, cat_table_hbm,
               item_out, cat_out, idx_v, cidx_v, *sems):
    wid = lax.axis_index("s") * _NC + lax.axis_index("c")
    base = wid * _B_PER_W
    pltpu.sync_copy(ids_hbm.at[pl.ds(base, _B_PER_W)], idx_v)
    pltpu.sync_copy(cids_hbm.at[pl.ds(base, _B_PER_W)], cidx_v)

    def body(g, carry):
        iv = idx_v[pl.ds(g * 16, 16)]
        cv = cidx_v[pl.ds(g * 16, 16)]
        for l in range(16):
            j = g * 16 + l
            pltpu.async_copy(item_table_hbm.at[pl.ds(iv[l], 1)],
                             item_out.at[pl.ds(base + j, 1)],
                             sems[l % _NSEM])
            pltpu.async_copy(cat_table_hbm.at[pl.ds(cv[l], 1)],
                             cat_out.at[pl.ds(base + j, 1)],
                             sems[l % _NSEM])
        return carry

    lax.fori_loop(0, _B_PER_W // 16, body, 0)
    per_sem = _B_PER_W // _NSEM
    for s in range(_NSEM):
        pltpu.make_async_copy(item_table_hbm.at[pl.ds(0, per_sem)],
                              item_out.at[pl.ds(base, per_sem)],
                              sems[s]).wait()
        pltpu.make_async_copy(cat_table_hbm.at[pl.ds(0, per_sem)],
                              cat_out.at[pl.ds(base, per_sem)],
                              sems[s]).wait()


_BM = 2048


def _dense_body(x1_ref, x2_ref, w1_ref, w2_ref, b_ref, o_ref):
    o_ref[...] = (
        jnp.dot(x1_ref[...], w1_ref[...], preferred_element_type=jnp.float32)
        + jnp.dot(x2_ref[...], w2_ref[...], preferred_element_type=jnp.float32)
        + b_ref[...]
    )


_tc_dense = pl.pallas_call(
    _dense_body,
    grid=(BATCH // _BM,),
    in_specs=[
        pl.BlockSpec((_BM, EMB), lambda i: (i, 0)),
        pl.BlockSpec((_BM, CAT_EMB), lambda i: (i, 0)),
        pl.BlockSpec((EMB, EMB), lambda i: (0, 0)),
        pl.BlockSpec((CAT_EMB, EMB), lambda i: (0, 0)),
        pl.BlockSpec((1, EMB), lambda i: (0, 0)),
    ],
    out_specs=pl.BlockSpec((_BM, EMB), lambda i: (i, 0)),
    out_shape=jax.ShapeDtypeStruct((BATCH, EMB), jnp.float32),
)


def kernel(item_id, item_category, item_table, cat_table, W, b):
    item_emb, cat_emb = _sc_gather(item_id, item_category, item_table,
                                   cat_table)
    return _tc_dense(item_emb, cat_emb, W[:EMB], W[EMB:], b.reshape(1, EMB))


# speedup vs baseline: 1.7550x; 1.7550x over previous
import functools

import jax
import jax.numpy as jnp
from jax import lax
from jax.experimental import pallas as pl
from jax.experimental.pallas import tpu as pltpu
from jax.experimental.pallas import tpu_sc as plsc

BATCH = 16384
EMB = 64
CAT_EMB = 32
CAT_VOCAB = 1000

_NC = 2
_NS = 16
_NW = _NC * _NS
_B_PER_W = BATCH // _NW
_ROUND = 64
_NR = _B_PER_W // _ROUND

_sc_mesh = plsc.VectorSubcoreMesh(core_axis_name="c", subcore_axis_name="s")


@functools.partial(
    pl.kernel,
    out_type=jax.ShapeDtypeStruct((BATCH * 8, EMB), jnp.float32),
    mesh=_sc_mesh,
    scratch_types=[
        pltpu.VMEM((_B_PER_W,), jnp.int32),
        pltpu.VMEM((_ROUND * 8, EMB), jnp.float32),
        pltpu.VMEM((_ROUND * 8, EMB), jnp.float32),
        pltpu.SemaphoreType.DMA,
    ],
)
def _sc_gather(tids_hbm, item_table_hbm, out, idx_v, tb_a, tb_b, sem):
    wid = lax.axis_index("s") * _NC + lax.axis_index("c")
    base = wid * _B_PER_W
    pltpu.sync_copy(tids_hbm.at[pl.ds(base, _B_PER_W)], idx_v)

    def one_round(r, tb):
        def body(g, carry):
            iv = idx_v[pl.ds(r * _ROUND + g * 16, 16)]
            for l in range(16):
                pltpu.async_copy(
                    item_table_hbm.at[pl.ds(iv[l] * 8, 8)],
                    tb.at[pl.ds((g * 16 + l) * 8, 8)], sem)
            return carry

        lax.fori_loop(0, _ROUND // 16, body, 0)
        pltpu.make_async_copy(item_table_hbm.at[pl.ds(0, _ROUND * 8)],
                              tb, sem).wait()
        pltpu.sync_copy(tb, out.at[pl.ds((base + r * _ROUND) * 8,
                                         _ROUND * 8)])

    def round_body(r, carry):
        one_round(r, tb_a)
        return carry

    lax.fori_loop(0, _NR, round_body, 0)


_BM = 2048


def _dense_body(ig_ref, oh_ref, cid_ref, ctab_ref, w1_ref, w2_ref, b_ref,
                o_ref):
    ig = ig_ref[...]
    oh = oh_ref[...]
    xi = jnp.zeros((_BM, EMB), jnp.float32)
    for k in range(8):
        xi = xi + ig[:, k, :] * oh[:, k][:, None]
    iota_c = lax.broadcasted_iota(jnp.int32, (_BM, CAT_VOCAB), 1)
    ohc = (cid_ref[...] == iota_c).astype(jnp.float32)
    xc = jnp.dot(ohc, ctab_ref[...], preferred_element_type=jnp.float32)
    o_ref[...] = (
        jnp.dot(xi, w1_ref[...], preferred_element_type=jnp.float32)
        + jnp.dot(xc, w2_ref[...], preferred_element_type=jnp.float32)
        + b_ref[...]
    )


_tc_dense = pl.pallas_call(
    _dense_body,
    grid=(BATCH // _BM,),
    in_specs=[
        pl.BlockSpec((_BM, 8, EMB), lambda i: (i, 0, 0)),
        pl.BlockSpec((_BM, 8), lambda i: (i, 0)),
        pl.BlockSpec((_BM, 1), lambda i: (i, 0)),
        pl.BlockSpec((CAT_VOCAB, CAT_EMB), lambda i: (0, 0)),
        pl.BlockSpec((EMB, EMB), lambda i: (0, 0)),
        pl.BlockSpec((CAT_EMB, EMB), lambda i: (0, 0)),
        pl.BlockSpec((1, EMB), lambda i: (0, 0)),
    ],
    out_specs=pl.BlockSpec((_BM, EMB), lambda i: (i, 0)),
    out_shape=jax.ShapeDtypeStruct((BATCH, EMB), jnp.float32),
)


def kernel(item_id, item_category, item_table, cat_table, W, b):
    sub8 = jnp.arange(8, dtype=jnp.int32)
    oh8 = ((item_id & 7)[:, None] == sub8).astype(jnp.float32)
    igath = _sc_gather(item_id >> 3, item_table)
    ig3 = igath.reshape(BATCH, 8, EMB)
    return _tc_dense(ig3, oh8, item_category[:, None], cat_table,
                     W[:EMB], W[EMB:], b.reshape(1, EMB))


# speedup vs baseline: 2.1672x; 1.2349x over previous
import functools

import jax
import jax.numpy as jnp
from jax import lax
from jax.experimental import pallas as pl
from jax.experimental.pallas import tpu as pltpu
from jax.experimental.pallas import tpu_sc as plsc

BATCH = 16384
EMB = 64
CAT_EMB = 32
LANES = 128

_NC = 2
_NS = 16
_NW = _NC * _NS
_CHUNK = 128
_B_PER_W = BATCH // _NW
_ROUND = 256

_sc_mesh = plsc.VectorSubcoreMesh(core_axis_name="c", subcore_axis_name="s")


@functools.partial(
    pl.kernel,
    out_type=[
        jax.ShapeDtypeStruct((BATCH, LANES), jnp.float32),
        jax.ShapeDtypeStruct((BATCH, LANES), jnp.float32),
    ],
    mesh=_sc_mesh,
    scratch_types=[
        pltpu.VMEM((_B_PER_W,), jnp.int32),
        pltpu.VMEM((_B_PER_W,), jnp.int32),
        pltpu.VMEM((_ROUND, LANES), jnp.float32),
        pltpu.VMEM((_ROUND, LANES), jnp.float32),
        pltpu.SemaphoreType.DMA,
    ],
)
def _sc_gather(ids_hbm, cids_hbm, itemp_hbm, catp_hbm,
               item_out, cat_out, idx_v, cidx_v, buf_a, buf_b, sem):
    wid = lax.axis_index("s") * _NC + lax.axis_index("c")
    base = wid * _B_PER_W
    pltpu.sync_copy(ids_hbm.at[pl.ds(base, _B_PER_W)], idx_v)
    pltpu.sync_copy(cids_hbm.at[pl.ds(base, _B_PER_W)], cidx_v)
    bufs = (buf_a, buf_b)
    for r in range(_B_PER_W // _ROUND):
        buf = bufs[r % 2]
        cps = [
            pltpu.async_copy(
                itemp_hbm.at[idx_v.at[pl.ds(r * _ROUND + k * _CHUNK, _CHUNK)]],
                buf.at[pl.ds(k * _CHUNK, _CHUNK)], sem)
            for k in range(_ROUND // _CHUNK)
        ]
        for cp in cps:
            cp.wait()
        pltpu.sync_copy(buf, item_out.at[pl.ds(base + r * _ROUND, _ROUND)])
    for r in range(_B_PER_W // _ROUND):
        buf = bufs[r % 2]
        cps = [
            pltpu.async_copy(
                catp_hbm.at[cidx_v.at[pl.ds(r * _ROUND + k * _CHUNK, _CHUNK)]],
                buf.at[pl.ds(k * _CHUNK, _CHUNK)], sem)
            for k in range(_ROUND // _CHUNK)
        ]
        for cp in cps:
            cp.wait()
        pltpu.sync_copy(buf, cat_out.at[pl.ds(base + r * _ROUND, _ROUND)])


_PC = 2048
_PGRID = 245
_PLINES = _PGRID * _PC


def _pairify_body(a_ref, b_ref, o_ref):
    eye = jnp.eye(EMB, dtype=jnp.float32)
    dn = (((0,), (0,)), ((), ()))
    at = lax.dot_general(a_ref[...], eye, dn,
                         preferred_element_type=jnp.float32)
    bt = lax.dot_general(b_ref[...], eye, dn,
                         preferred_element_type=jnp.float32)
    o_ref[...] = jnp.concatenate([at, bt], axis=1)


_tc_pairify = pl.pallas_call(
    _pairify_body,
    grid=(_PGRID,),
    in_specs=[
        pl.BlockSpec((EMB, _PC),
                     lambda i: (0, jnp.where(i == _PGRID - 1,
                                             2 * i - 1, 2 * i))),
        pl.BlockSpec((EMB, _PC),
                     lambda i: (0, jnp.minimum(2 * i + 1, 2 * _PGRID - 2))),
    ],
    out_specs=pl.BlockSpec((_PC, 2 * EMB), lambda i: (i, 0)),
    out_shape=jax.ShapeDtypeStruct((_PLINES, 2 * EMB), jnp.float32),
)


_BM = 2048


def _dense_body(ip_ref, cp_ref, pi_ref, oh_ref, w1_ref, w2_ref, b_ref,
                o_ref):
    ip = ip_ref[...]
    cp = cp_ref[...]
    pi = pi_ref[...]
    oh = oh_ref[...]
    xi = ip[:, :EMB] * (1.0 - pi) + ip[:, EMB:] * pi
    xc = jnp.zeros((_BM, CAT_EMB), jnp.float32)
    for q in range(4):
        xc = xc + cp[:, q * CAT_EMB:(q + 1) * CAT_EMB] * oh[:, q][:, None]
    o_ref[...] = (
        jnp.dot(xi, w1_ref[...], preferred_element_type=jnp.float32)
        + jnp.dot(xc, w2_ref[...], preferred_element_type=jnp.float32)
        + b_ref[...]
    )


_tc_dense = pl.pallas_call(
    _dense_body,
    grid=(BATCH // _BM,),
    in_specs=[
        pl.BlockSpec((_BM, LANES), lambda i: (i, 0)),
        pl.BlockSpec((_BM, LANES), lambda i: (i, 0)),
        pl.BlockSpec((_BM, 1), lambda i: (i, 0)),
        pl.BlockSpec((_BM, 4), lambda i: (i, 0)),
        pl.BlockSpec((EMB, EMB), lambda i: (0, 0)),
        pl.BlockSpec((CAT_EMB, EMB), lambda i: (0, 0)),
        pl.BlockSpec((1, EMB), lambda i: (0, 0)),
    ],
    out_specs=pl.BlockSpec((_BM, EMB), lambda i: (i, 0)),
    out_shape=jax.ShapeDtypeStruct((BATCH, EMB), jnp.float32),
)


def kernel(item_id, item_category, item_table, cat_table, W, b):
    table_t = item_table.T
    itemp = _tc_pairify(table_t, table_t)
    catp = cat_table.reshape(cat_table.shape[0] // 4, 4 * CAT_EMB)
    ids_half = ((item_id >> 12) << 11) | (item_id & 2047)
    cids_quarter = item_category >> 2
    par_i = (((item_id >> 11) & 1) | (item_id >= 999424)
             ).astype(jnp.float32)[:, None]
    oh_c = ((item_category & 3)[:, None]
            == jnp.arange(4, dtype=jnp.int32)).astype(jnp.float32)
    ipair, cpair = _sc_gather(ids_half, cids_quarter, itemp, catp)
    return _tc_dense(ipair, cpair, par_i, oh_c, W[:EMB], W[EMB:],
                     b.reshape(1, EMB))


# speedup vs baseline: 2.1706x; 1.0016x over previous
import functools

import jax
import jax.numpy as jnp
from jax import lax
from jax.experimental import pallas as pl
from jax.experimental.pallas import tpu as pltpu
from jax.experimental.pallas import tpu_sc as plsc

BATCH = 16384
EMB = 64
CAT_EMB = 32
LANES = 128

_NC = 2
_NS = 16
_NW = _NC * _NS
_CHUNK = 128
_B_PER_W = BATCH // _NW
_ROUND = 256

_sc_mesh = plsc.VectorSubcoreMesh(core_axis_name="c", subcore_axis_name="s")


@functools.partial(
    pl.kernel,
    out_type=[
        jax.ShapeDtypeStruct((BATCH, LANES), jnp.float32),
        jax.ShapeDtypeStruct((BATCH, LANES), jnp.float32),
    ],
    mesh=_sc_mesh,
    scratch_types=[
        pltpu.VMEM((_B_PER_W,), jnp.int32),
        pltpu.VMEM((_B_PER_W,), jnp.int32),
        pltpu.VMEM((_ROUND, LANES), jnp.float32),
        pltpu.VMEM((_ROUND, LANES), jnp.float32),
        pltpu.SemaphoreType.DMA,
    ],
)
def _sc_gather(ids_hbm, cids_hbm, itemp_hbm, catp_hbm,
               item_out, cat_out, idx_v, cidx_v, buf_a, buf_b, sem):
    wid = lax.axis_index("s") * _NC + lax.axis_index("c")
    base = wid * _B_PER_W
    pltpu.sync_copy(ids_hbm.at[pl.ds(base, _B_PER_W)], idx_v)
    pltpu.sync_copy(cids_hbm.at[pl.ds(base, _B_PER_W)], cidx_v)
    bufs = (buf_a, buf_b)
    for r in range(_B_PER_W // _ROUND):
        buf = bufs[r % 2]
        cps = [
            pltpu.async_copy(
                itemp_hbm.at[idx_v.at[pl.ds(r * _ROUND + k * _CHUNK, _CHUNK)]],
                buf.at[pl.ds(k * _CHUNK, _CHUNK)], sem)
            for k in range(_ROUND // _CHUNK)
        ]
        for cp in cps:
            cp.wait()
        pltpu.sync_copy(buf, item_out.at[pl.ds(base + r * _ROUND, _ROUND)])
    for r in range(_B_PER_W // _ROUND):
        buf = bufs[r % 2]
        cps = [
            pltpu.async_copy(
                catp_hbm.at[cidx_v.at[pl.ds(r * _ROUND + k * _CHUNK, _CHUNK)]],
                buf.at[pl.ds(k * _CHUNK, _CHUNK)], sem)
            for k in range(_ROUND // _CHUNK)
        ]
        for cp in cps:
            cp.wait()
        pltpu.sync_copy(buf, cat_out.at[pl.ds(base + r * _ROUND, _ROUND)])


_PC = 2048
_PGRID = 245
_PLINES = _PGRID * _PC


def _pairify_body(a_ref, b_ref, o_ref):
    o_ref[...] = jnp.concatenate([a_ref[...].T, b_ref[...].T], axis=1)


_tc_pairify = pl.pallas_call(
    _pairify_body,
    grid=(_PGRID,),
    in_specs=[
        pl.BlockSpec((EMB, _PC),
                     lambda i: (0, jnp.where(i == _PGRID - 1,
                                             2 * i - 1, 2 * i))),
        pl.BlockSpec((EMB, _PC),
                     lambda i: (0, jnp.minimum(2 * i + 1, 2 * _PGRID - 2))),
    ],
    out_specs=pl.BlockSpec((_PC, 2 * EMB), lambda i: (i, 0)),
    out_shape=jax.ShapeDtypeStruct((_PLINES, 2 * EMB), jnp.float32),
)


_BM = 2048


def _dense_body(ip_ref, cp_ref, pi_ref, oh_ref, w1_ref, w2_ref, b_ref,
                o_ref):
    ip = ip_ref[...]
    cp = cp_ref[...]
    pi = pi_ref[...]
    oh = oh_ref[...]
    xi = ip[:, :EMB] * (1.0 - pi) + ip[:, EMB:] * pi
    xc = jnp.zeros((_BM, CAT_EMB), jnp.float32)
    for q in range(4):
        xc = xc + cp[:, q * CAT_EMB:(q + 1) * CAT_EMB] * oh[:, q][:, None]
    o_ref[...] = (
        jnp.dot(xi, w1_ref[...], preferred_element_type=jnp.float32)
        + jnp.dot(xc, w2_ref[...], preferred_element_type=jnp.float32)
        + b_ref[...]
    )


_tc_dense = pl.pallas_call(
    _dense_body,
    grid=(BATCH // _BM,),
    in_specs=[
        pl.BlockSpec((_BM, LANES), lambda i: (i, 0)),
        pl.BlockSpec((_BM, LANES), lambda i: (i, 0)),
        pl.BlockSpec((_BM, 1), lambda i: (i, 0)),
        pl.BlockSpec((_BM, 4), lambda i: (i, 0)),
        pl.BlockSpec((EMB, EMB), lambda i: (0, 0)),
        pl.BlockSpec((CAT_EMB, EMB), lambda i: (0, 0)),
        pl.BlockSpec((1, EMB), lambda i: (0, 0)),
    ],
    out_specs=pl.BlockSpec((_BM, EMB), lambda i: (i, 0)),
    out_shape=jax.ShapeDtypeStruct((BATCH, EMB), jnp.float32),
)


def kernel(item_id, item_category, item_table, cat_table, W, b):
    table_t = item_table.T
    itemp = _tc_pairify(table_t, table_t)
    catp = cat_table.reshape(cat_table.shape[0] // 4, 4 * CAT_EMB)
    ids_half = ((item_id >> 12) << 11) | (item_id & 2047)
    cids_quarter = item_category >> 2
    par_i = (((item_id >> 11) & 1) | (item_id >= 999424)
             ).astype(jnp.float32)[:, None]
    oh_c = ((item_category & 3)[:, None]
            == jnp.arange(4, dtype=jnp.int32)).astype(jnp.float32)
    ipair, cpair = _sc_gather(ids_half, cids_quarter, itemp, catp)
    return _tc_dense(ipair, cpair, par_i, oh_c, W[:EMB], W[EMB:],
                     b.reshape(1, EMB))


# speedup vs baseline: 2.8517x; 1.3138x over previous
import functools

import jax
import jax.numpy as jnp
from jax import lax
from jax.experimental import pallas as pl
from jax.experimental.pallas import tpu as pltpu
from jax.experimental.pallas import tpu_sc as plsc

BATCH = 16384
EMB = 64
CAT_EMB = 32
LANES = 128

_NC = 2
_NS = 16
_NW = _NC * _NS
_CHUNK = 128
_B_PER_W = BATCH // _NW
_ROUND = 256

_sc_mesh = plsc.VectorSubcoreMesh(core_axis_name="c", subcore_axis_name="s")


@functools.partial(
    pl.kernel,
    out_type=[
        jax.ShapeDtypeStruct((BATCH, LANES), jnp.float32),
        jax.ShapeDtypeStruct((BATCH, LANES), jnp.float32),
    ],
    mesh=_sc_mesh,
    scratch_types=[
        pltpu.VMEM((_B_PER_W,), jnp.int32),
        pltpu.VMEM((_B_PER_W,), jnp.int32),
        pltpu.VMEM((_ROUND, LANES), jnp.float32),
        pltpu.VMEM((_ROUND, LANES), jnp.float32),
        pltpu.SemaphoreType.DMA,
    ],
)
def _sc_gather(ids_hbm, cids_hbm, itemp_hbm, catp_hbm,
               item_out, cat_out, idx_v, cidx_v, buf_a, buf_b, sem):
    wid = lax.axis_index("s") * _NC + lax.axis_index("c")
    base = wid * _B_PER_W
    pltpu.sync_copy(ids_hbm.at[pl.ds(base, _B_PER_W)], idx_v)
    pltpu.sync_copy(cids_hbm.at[pl.ds(base, _B_PER_W)], cidx_v)
    bufs = (buf_a, buf_b)
    for r in range(_B_PER_W // _ROUND):
        buf = bufs[r % 2]
        cps = [
            pltpu.async_copy(
                itemp_hbm.at[idx_v.at[pl.ds(r * _ROUND + k * _CHUNK, _CHUNK)]],
                buf.at[pl.ds(k * _CHUNK, _CHUNK)], sem)
            for k in range(_ROUND // _CHUNK)
        ]
        for cp in cps:
            cp.wait()
        pltpu.sync_copy(buf, item_out.at[pl.ds(base + r * _ROUND, _ROUND)])
    for r in range(_B_PER_W // _ROUND):
        buf = bufs[r % 2]
        cps = [
            pltpu.async_copy(
                catp_hbm.at[cidx_v.at[pl.ds(r * _ROUND + k * _CHUNK, _CHUNK)]],
                buf.at[pl.ds(k * _CHUNK, _CHUNK)], sem)
            for k in range(_ROUND // _CHUNK)
        ]
        for cp in cps:
            cp.wait()
        pltpu.sync_copy(buf, cat_out.at[pl.ds(base + r * _ROUND, _ROUND)])


_PC = 8192
_PSH = 13
_PGRID = 62
_PLINES = _PGRID * _PC
_PTAIL = (2 * _PGRID - 2) * _PC


def _pairify_body(a_ref, b_ref, o_ref):
    o_ref[...] = jnp.concatenate([a_ref[...].T, b_ref[...].T], axis=1)


_tc_pairify = pl.pallas_call(
    _pairify_body,
    grid=(_PGRID,),
    in_specs=[
        pl.BlockSpec((EMB, _PC),
                     lambda i: (0, jnp.where(i == _PGRID - 1,
                                             2 * i - 1, 2 * i))),
        pl.BlockSpec((EMB, _PC),
                     lambda i: (0, jnp.minimum(2 * i + 1, 2 * _PGRID - 2))),
    ],
    out_specs=pl.BlockSpec((_PC, 2 * EMB), lambda i: (i, 0)),
    out_shape=jax.ShapeDtypeStruct((_PLINES, 2 * EMB), jnp.float32),
)


_BM = 2048


def _dense_body(ip_ref, cp_ref, pi_ref, oh_ref, w1_ref, w2_ref, b_ref,
                o_ref):
    ip = ip_ref[...]
    cp = cp_ref[...]
    pi = pi_ref[...]
    oh = oh_ref[...]
    xi = ip[:, :EMB] * (1.0 - pi) + ip[:, EMB:] * pi
    xc = jnp.zeros((_BM, CAT_EMB), jnp.float32)
    for q in range(4):
        xc = xc + cp[:, q * CAT_EMB:(q + 1) * CAT_EMB] * oh[:, q][:, None]
    o_ref[...] = (
        jnp.dot(xi, w1_ref[...], preferred_element_type=jnp.float32)
        + jnp.dot(xc, w2_ref[...], preferred_element_type=jnp.float32)
        + b_ref[...]
    )


_tc_dense = pl.pallas_call(
    _dense_body,
    grid=(BATCH // _BM,),
    in_specs=[
        pl.BlockSpec((_BM, LANES), lambda i: (i, 0)),
        pl.BlockSpec((_BM, LANES), lambda i: (i, 0)),
        pl.BlockSpec((_BM, 1), lambda i: (i, 0)),
        pl.BlockSpec((_BM, 4), lambda i: (i, 0)),
        pl.BlockSpec((EMB, EMB), lambda i: (0, 0)),
        pl.BlockSpec((CAT_EMB, EMB), lambda i: (0, 0)),
        pl.BlockSpec((1, EMB), lambda i: (0, 0)),
    ],
    out_specs=pl.BlockSpec((_BM, EMB), lambda i: (i, 0)),
    out_shape=jax.ShapeDtypeStruct((BATCH, EMB), jnp.float32),
)


def kernel(item_id, item_category, item_table, cat_table, W, b):
    table_t = item_table.T
    itemp = _tc_pairify(table_t, table_t)
    catp = cat_table.reshape(cat_table.shape[0] // 4, 4 * CAT_EMB)
    ids_half = ((item_id >> (_PSH + 1)) << _PSH) | (item_id & (_PC - 1))
    cids_quarter = item_category >> 2
    par_i = (((item_id >> _PSH) & 1) | (item_id >= _PTAIL)
             ).astype(jnp.float32)[:, None]
    oh_c = ((item_category & 3)[:, None]
            == jnp.arange(4, dtype=jnp.int32)).astype(jnp.float32)
    ipair, cpair = _sc_gather(ids_half, cids_quarter, itemp, catp)
    return _tc_dense(ipair, cpair, par_i, oh_c, W[:EMB], W[EMB:],
                     b.reshape(1, EMB))


# speedup vs baseline: 3.0075x; 1.0546x over previous
import functools

import jax
import jax.numpy as jnp
from jax import lax
from jax.experimental import pallas as pl
from jax.experimental.pallas import tpu as pltpu
from jax.experimental.pallas import tpu_sc as plsc

BATCH = 16384
EMB = 64
CAT_EMB = 32
LANES = 128

_NC = 2
_NS = 16
_NW = _NC * _NS
_CHUNK = 128
_B_PER_W = BATCH // _NW
_ROUND = 256

_sc_mesh = plsc.VectorSubcoreMesh(core_axis_name="c", subcore_axis_name="s")


@functools.partial(
    pl.kernel,
    out_type=[
        jax.ShapeDtypeStruct((BATCH, LANES), jnp.float32),
        jax.ShapeDtypeStruct((BATCH, LANES), jnp.float32),
    ],
    mesh=_sc_mesh,
    scratch_types=[
        pltpu.VMEM((_B_PER_W,), jnp.int32),
        pltpu.VMEM((_B_PER_W,), jnp.int32),
        pltpu.VMEM((_ROUND, LANES), jnp.float32),
        pltpu.VMEM((_ROUND, LANES), jnp.float32),
        pltpu.SemaphoreType.DMA,
    ],
)
def _sc_gather(ids_hbm, cids_hbm, itemp_hbm, catp_hbm,
               item_out, cat_out, idx_v, cidx_v, buf_a, buf_b, sem):
    wid = lax.axis_index("s") * _NC + lax.axis_index("c")
    base = wid * _B_PER_W
    pltpu.sync_copy(ids_hbm.at[pl.ds(base, _B_PER_W)], idx_v)
    pltpu.sync_copy(cids_hbm.at[pl.ds(base, _B_PER_W)], cidx_v)
    bufs = (buf_a, buf_b)
    for r in range(_B_PER_W // _ROUND):
        buf = bufs[r % 2]
        cps = [
            pltpu.async_copy(
                itemp_hbm.at[idx_v.at[pl.ds(r * _ROUND + k * _CHUNK, _CHUNK)]],
                buf.at[pl.ds(k * _CHUNK, _CHUNK)], sem)
            for k in range(_ROUND // _CHUNK)
        ]
        for cp in cps:
            cp.wait()
        pltpu.sync_copy(buf, item_out.at[pl.ds(base + r * _ROUND, _ROUND)])
    for r in range(_B_PER_W // _ROUND):
        buf = bufs[r % 2]
        cps = [
            pltpu.async_copy(
                catp_hbm.at[cidx_v.at[pl.ds(r * _ROUND + k * _CHUNK, _CHUNK)]],
                buf.at[pl.ds(k * _CHUNK, _CHUNK)], sem)
            for k in range(_ROUND // _CHUNK)
        ]
        for cp in cps:
            cp.wait()
        pltpu.sync_copy(buf, cat_out.at[pl.ds(base + r * _ROUND, _ROUND)])


_PC = 16384
_PSH = 14
_PGRID = 31
_PLINES = _PGRID * _PC
_PTAIL = (2 * _PGRID - 2) * _PC


def _pairify_body(a_ref, b_ref, o_ref):
    o_ref[...] = jnp.concatenate([a_ref[...].T, b_ref[...].T], axis=1)


_tc_pairify = pl.pallas_call(
    _pairify_body,
    grid=(_PGRID,),
    in_specs=[
        pl.BlockSpec((EMB, _PC), lambda i: (0, 2 * i)),
        pl.BlockSpec((EMB, _PC), lambda i: (0, 2 * i + 1)),
    ],
    out_specs=pl.BlockSpec((_PC, 2 * EMB), lambda i: (i, 0)),
    out_shape=jax.ShapeDtypeStruct((_PLINES, 2 * EMB), jnp.float32),
)


_BM = 2048


def _dense_body(ip_ref, cp_ref, pi_ref, oh_ref, w1_ref, w2_ref, b_ref,
                o_ref):
    ip = ip_ref[...]
    cp = cp_ref[...]
    pi = pi_ref[...]
    oh = oh_ref[...]
    xi = ip[:, :EMB] * (1.0 - pi) + ip[:, EMB:] * pi
    xc = jnp.zeros((_BM, CAT_EMB), jnp.float32)
    for q in range(4):
        xc = xc + cp[:, q * CAT_EMB:(q + 1) * CAT_EMB] * oh[:, q][:, None]
    o_ref[...] = (
        jnp.dot(xi, w1_ref[...], preferred_element_type=jnp.float32)
        + jnp.dot(xc, w2_ref[...], preferred_element_type=jnp.float32)
        + b_ref[...]
    )


_tc_dense = pl.pallas_call(
    _dense_body,
    grid=(BATCH // _BM,),
    in_specs=[
        pl.BlockSpec((_BM, LANES), lambda i: (i, 0)),
        pl.BlockSpec((_BM, LANES), lambda i: (i, 0)),
        pl.BlockSpec((_BM, 1), lambda i: (i, 0)),
        pl.BlockSpec((_BM, 4), lambda i: (i, 0)),
        pl.BlockSpec((EMB, EMB), lambda i: (0, 0)),
        pl.BlockSpec((CAT_EMB, EMB), lambda i: (0, 0)),
        pl.BlockSpec((1, EMB), lambda i: (0, 0)),
    ],
    out_specs=pl.BlockSpec((_BM, EMB), lambda i: (i, 0)),
    out_shape=jax.ShapeDtypeStruct((BATCH, EMB), jnp.float32),
)


def kernel(item_id, item_category, item_table, cat_table, W, b):
    table_t = item_table.T
    itemp = _tc_pairify(table_t, table_t)
    catp = cat_table.reshape(cat_table.shape[0] // 4, 4 * CAT_EMB)
    ids_half = ((item_id >> (_PSH + 1)) << _PSH) | (item_id & (_PC - 1))
    cids_quarter = item_category >> 2
    par_i = ((item_id >> _PSH) & 1).astype(jnp.float32)[:, None]
    oh_c = ((item_category & 3)[:, None]
            == jnp.arange(4, dtype=jnp.int32)).astype(jnp.float32)
    ipair, cpair = _sc_gather(ids_half, cids_quarter, itemp, catp)
    return _tc_dense(ipair, cpair, par_i, oh_c, W[:EMB], W[EMB:],
                     b.reshape(1, EMB))
